# baseline (speedup 1.0000x reference)
import functools

import numpy as np
import jax
import jax.numpy as jnp
from jax.experimental import pallas as pl
from jax.experimental.pallas import tpu as pltpu


def _interp_consts(n_in, n_out):
    scale = np.float32(np.float32(n_in - 1) / np.float32(n_out - 1))
    pos = np.arange(n_out, dtype=np.float32) * scale
    lo = np.clip(np.floor(pos), 0, n_in - 1).astype(np.int32)
    hi = np.minimum(lo + 1, n_in - 1)
    frac = (pos - lo.astype(np.float32)).astype(np.float32)
    return lo, hi, frac


def _interp_matrix_np(n_in, n_out):
    lo, hi, frac = _interp_consts(n_in, n_out)
    a = np.zeros((n_out, n_in), np.float32)
    for i in range(n_out):
        a[i, lo[i]] += 1.0 - frac[i]
        a[i, hi[i]] += frac[i]
    return a


def _fused_kernel(awt_ref, wu_ref, w1_ref, w2_ref, sh1_ref, sh2_ref,
                  ml_ref, mr_ref, x_ref, o_ref,
                  xw_ref, up_ref, ypad_ref, hpad_ref,
                  *, hin, win, wo, ho, m, padl, g, lo, hi, frac):
    cin = wu_ref.shape[2] // 3

    for ref in (up_ref, ypad_ref, hpad_ref):
        ref[:, :padl] = jnp.zeros_like(ref[:, :padl])
        ref[:, padl + m:] = jnp.zeros_like(ref[:, padl + m:])

    xsrc = x_ref[0]
    for k in range(0, hin, g):
        seg = xsrc[:, k:k + g, :].reshape(cin, g * win).astype(jnp.bfloat16)
        blk = jnp.dot(seg, awt_ref[...],
                      preferred_element_type=jnp.float32)
        xw_ref[:, k * wo:(k + g) * wo] = blk.astype(jnp.bfloat16)
    for i in range(ho):
        l, h, f = lo[i], hi[i], frac[i]
        r0 = xw_ref[:, l * wo:(l + 1) * wo].astype(jnp.float32)
        if f == 0.0 or h == l:
            row = r0
        else:
            r1 = xw_ref[:, h * wo:(h + 1) * wo].astype(jnp.float32)
            row = r0 * (1.0 - f) + r1 * f
        up_ref[cin:2 * cin, padl + i * wo: padl + (i + 1) * wo] = \
            row.astype(jnp.bfloat16)

    mask_l = ml_ref[...]
    mask_r = mr_ref[...]

    def conv3x3(p3_ref, w3_ref):
        c = w3_ref.shape[2] // 3
        a, b = padl - wo, padl + m + wo
        p3_ref[:c, a:b] = p3_ref[c:2 * c, a - 1:b - 1] * mask_l
        p3_ref[2 * c:3 * c, a:b] = p3_ref[c:2 * c, a + 1:b + 1] * mask_r
        acc = None
        for dy in range(3):
            s = padl + (dy - 1) * wo
            part = jnp.dot(w3_ref[dy], p3_ref[:, s:s + m],
                           preferred_element_type=jnp.float32)
            acc = part if acc is None else acc + part
        return acc

    c2 = w1_ref.shape[1]
    y = jnp.maximum(conv3x3(up_ref, wu_ref), 0.0)
    ypad_ref[c2:2 * c2, padl:padl + m] = y.astype(jnp.bfloat16)
    h1 = jnp.maximum(conv3x3(ypad_ref, w1_ref) + sh1_ref[...], 0.0)
    hpad_ref[c2:2 * c2, padl:padl + m] = h1.astype(jnp.bfloat16)
    out = jnp.maximum(conv3x3(hpad_ref, w2_ref) + sh2_ref[...] + y, 0.0)
    o_ref[0] = out.astype(o_ref.dtype)


def kernel(x, w_up, w1, b1, bn1_gamma, bn1_beta, bn1_mean, bn1_var,
           w2, b2, bn2_gamma, bn2_beta, bn2_mean, bn2_var):
    eps = 1e-5
    n, cin, hin, win = x.shape
    c2 = w_up.shape[0]
    ho, wo = 2 * hin, 2 * win
    m = ho * wo
    padl = ((wo + 1 + 127) // 128) * 128
    mp = m + 2 * padl

    s1 = bn1_gamma / jnp.sqrt(bn1_var + eps)
    sh1 = ((b1 - bn1_mean) * s1 + bn1_beta).reshape(c2, 1).astype(jnp.float32)
    s2 = bn2_gamma / jnp.sqrt(bn2_var + eps)
    sh2 = ((b2 - bn2_mean) * s2 + bn2_beta).reshape(c2, 1).astype(jnp.float32)

    def tap_major(w, scale=None):
        if scale is not None:
            w = w * scale[:, None, None, None]
        o, i = w.shape[0], w.shape[1]
        return jnp.transpose(w, (2, 0, 3, 1)).reshape(3, o, 3 * i) \
            .astype(jnp.bfloat16)

    wu9 = tap_major(w_up)
    w19 = tap_major(w1, s1)
    w29 = tap_major(w2, s2)

    g = max(1, 256 // win)
    while hin % g:
        g //= 2
    aw1 = _interp_matrix_np(win, wo).T
    awb = np.zeros((g * win, g * wo), np.float32)
    for j in range(g):
        awb[j * win:(j + 1) * win, j * wo:(j + 1) * wo] = aw1
    awt = jnp.asarray(awb, jnp.bfloat16)
    lo, hi, frac = _interp_consts(hin, ho)

    col = jnp.arange(m + 2 * wo, dtype=jnp.int32) % wo
    mask_l = (col >= 1).astype(jnp.bfloat16).reshape(1, m + 2 * wo)
    mask_r = (col < wo - 1).astype(jnp.bfloat16).reshape(1, m + 2 * wo)

    flops = 2 * n * m * 9 * (c2 * cin + 2 * c2 * c2) + 4 * n * cin * m
    bytes_accessed = 4 * (n * cin * hin * win + n * c2 * m) \
        + 2 * 9 * (c2 * cin + 2 * c2 * c2) + 2 * 2 * m

    out = pl.pallas_call(
        functools.partial(_fused_kernel, hin=hin, win=win, wo=wo, ho=ho, m=m,
                          padl=padl, g=g,
                          lo=tuple(int(v) for v in lo),
                          hi=tuple(int(v) for v in hi),
                          frac=tuple(float(v) for v in frac)),
        out_shape=jax.ShapeDtypeStruct((n, c2, m), x.dtype),
        grid_spec=pltpu.PrefetchScalarGridSpec(
            num_scalar_prefetch=0,
            grid=(n,),
            in_specs=[
                pl.BlockSpec((g * win, g * wo), lambda i: (0, 0)),
                pl.BlockSpec((3, c2, 3 * cin), lambda i: (0, 0, 0)),
                pl.BlockSpec((3, c2, 3 * c2), lambda i: (0, 0, 0)),
                pl.BlockSpec((3, c2, 3 * c2), lambda i: (0, 0, 0)),
                pl.BlockSpec((c2, 1), lambda i: (0, 0)),
                pl.BlockSpec((c2, 1), lambda i: (0, 0)),
                pl.BlockSpec((1, m + 2 * wo), lambda i: (0, 0)),
                pl.BlockSpec((1, m + 2 * wo), lambda i: (0, 0)),
                pl.BlockSpec((1, cin, hin, win), lambda i: (i, 0, 0, 0)),
            ],
            out_specs=pl.BlockSpec((1, c2, m), lambda i: (i, 0, 0)),
            scratch_shapes=[
                pltpu.VMEM((cin, hin * wo), jnp.bfloat16),
                pltpu.VMEM((3 * cin, mp), jnp.bfloat16),
                pltpu.VMEM((3 * c2, mp), jnp.bfloat16),
                pltpu.VMEM((3 * c2, mp), jnp.bfloat16),
            ],
        ),
        compiler_params=pltpu.CompilerParams(
            dimension_semantics=("parallel",)),
        cost_estimate=pl.CostEstimate(
            flops=flops, transcendentals=0, bytes_accessed=bytes_accessed),
    )(awt, wu9, w19, w29, sh1, sh2, mask_l, mask_r, x)
    return out.reshape(n, c2, ho, wo)


# speedup vs baseline: 1.1266x; 1.1266x over previous
import functools

import numpy as np
import jax
import jax.numpy as jnp
from jax.experimental import pallas as pl
from jax.experimental.pallas import tpu as pltpu


def _interp_consts(n_in, n_out):
    scale = np.float32(np.float32(n_in - 1) / np.float32(n_out - 1))
    pos = np.arange(n_out, dtype=np.float32) * scale
    lo = np.clip(np.floor(pos), 0, n_in - 1).astype(np.int32)
    hi = np.minimum(lo + 1, n_in - 1)
    frac = (pos - lo.astype(np.float32)).astype(np.float32)
    return lo, hi, frac


def _interp_matrix_np(n_in, n_out):
    lo, hi, frac = _interp_consts(n_in, n_out)
    a = np.zeros((n_out, n_in), np.float32)
    for i in range(n_out):
        a[i, lo[i]] += 1.0 - frac[i]
        a[i, hi[i]] += frac[i]
    return a


def _fused_kernel(awt_ref, wu_ref, w1_ref, w2_ref, sh1_ref, sh2_ref,
                  ml_ref, mr_ref, x_ref, o_ref,
                  xw_ref, up_ref, ypad_ref, hpad_ref,
                  *, hin, win, wo, ho, m, padl, g, lo, hi, frac):
    cin = wu_ref.shape[2] // 3

    for ref in (up_ref, ypad_ref, hpad_ref):
        ref[:, :padl] = jnp.zeros_like(ref[:, :padl])
        ref[:, padl + m:] = jnp.zeros_like(ref[:, padl + m:])

    xsrc = x_ref[0].astype(jnp.bfloat16)
    for k in range(0, hin, g):
        blk = jnp.dot(xsrc[:, k * win:(k + g) * win], awt_ref[...],
                      preferred_element_type=jnp.float32)
        xw_ref[:, k * wo:(k + g) * wo] = blk.astype(jnp.bfloat16)
    for i in range(ho):
        l, h, f = lo[i], hi[i], frac[i]
        r0 = xw_ref[:, l * wo:(l + 1) * wo].astype(jnp.float32)
        if f == 0.0 or h == l:
            row = r0
        else:
            r1 = xw_ref[:, h * wo:(h + 1) * wo].astype(jnp.float32)
            row = r0 * (1.0 - f) + r1 * f
        up_ref[cin:2 * cin, padl + i * wo: padl + (i + 1) * wo] = \
            row.astype(jnp.bfloat16)

    mask_l = ml_ref[...]
    mask_r = mr_ref[...]

    def conv3x3(p3_ref, w3_ref):
        c = w3_ref.shape[2] // 3
        a, b = padl - wo, padl + m + wo
        p3_ref[:c, a:b] = p3_ref[c:2 * c, a - 1:b - 1] * mask_l
        p3_ref[2 * c:3 * c, a:b] = p3_ref[c:2 * c, a + 1:b + 1] * mask_r
        acc = None
        for dy in range(3):
            s = padl + (dy - 1) * wo
            part = jnp.dot(w3_ref[dy], p3_ref[:, s:s + m],
                           preferred_element_type=jnp.float32)
            acc = part if acc is None else acc + part
        return acc

    c2 = w1_ref.shape[1]
    y = jnp.maximum(conv3x3(up_ref, wu_ref), 0.0)
    ypad_ref[c2:2 * c2, padl:padl + m] = y.astype(jnp.bfloat16)
    h1 = jnp.maximum(conv3x3(ypad_ref, w1_ref) + sh1_ref[...], 0.0)
    hpad_ref[c2:2 * c2, padl:padl + m] = h1.astype(jnp.bfloat16)
    out = jnp.maximum(conv3x3(hpad_ref, w2_ref) + sh2_ref[...] + y, 0.0)
    o_ref[0] = out.astype(o_ref.dtype)


def kernel(x, w_up, w1, b1, bn1_gamma, bn1_beta, bn1_mean, bn1_var,
           w2, b2, bn2_gamma, bn2_beta, bn2_mean, bn2_var):
    eps = 1e-5
    n, cin, hin, win = x.shape
    c2 = w_up.shape[0]
    ho, wo = 2 * hin, 2 * win
    m = ho * wo
    padl = ((wo + 1 + 127) // 128) * 128
    mp = m + 2 * padl

    s1 = bn1_gamma / jnp.sqrt(bn1_var + eps)
    sh1 = ((b1 - bn1_mean) * s1 + bn1_beta).reshape(c2, 1).astype(jnp.float32)
    s2 = bn2_gamma / jnp.sqrt(bn2_var + eps)
    sh2 = ((b2 - bn2_mean) * s2 + bn2_beta).reshape(c2, 1).astype(jnp.float32)

    def tap_major(w, scale=None):
        if scale is not None:
            w = w * scale[:, None, None, None]
        o, i = w.shape[0], w.shape[1]
        return jnp.transpose(w, (2, 0, 3, 1)).reshape(3, o, 3 * i) \
            .astype(jnp.bfloat16)

    wu9 = tap_major(w_up)
    w19 = tap_major(w1, s1)
    w29 = tap_major(w2, s2)

    g = max(1, 256 // win)
    while hin % g:
        g //= 2
    aw1 = _interp_matrix_np(win, wo).T
    awb = np.zeros((g * win, g * wo), np.float32)
    for j in range(g):
        awb[j * win:(j + 1) * win, j * wo:(j + 1) * wo] = aw1
    awt = jnp.asarray(awb, jnp.bfloat16)
    lo, hi, frac = _interp_consts(hin, ho)

    col = jnp.arange(m + 2 * wo, dtype=jnp.int32) % wo
    mask_l = (col >= 1).astype(jnp.bfloat16).reshape(1, m + 2 * wo)
    mask_r = (col < wo - 1).astype(jnp.bfloat16).reshape(1, m + 2 * wo)

    x2 = x.reshape(n, cin, hin * win)

    flops = 2 * n * m * 9 * (c2 * cin + 2 * c2 * c2) + 4 * n * cin * m
    bytes_accessed = 4 * (n * cin * hin * win + n * c2 * m) \
        + 2 * 9 * (c2 * cin + 2 * c2 * c2) + 2 * 2 * m

    out = pl.pallas_call(
        functools.partial(_fused_kernel, hin=hin, win=win, wo=wo, ho=ho, m=m,
                          padl=padl, g=g,
                          lo=tuple(int(v) for v in lo),
                          hi=tuple(int(v) for v in hi),
                          frac=tuple(float(v) for v in frac)),
        out_shape=jax.ShapeDtypeStruct((n, c2, m), x.dtype),
        grid_spec=pltpu.PrefetchScalarGridSpec(
            num_scalar_prefetch=0,
            grid=(n,),
            in_specs=[
                pl.BlockSpec((g * win, g * wo), lambda i: (0, 0)),
                pl.BlockSpec((3, c2, 3 * cin), lambda i: (0, 0, 0)),
                pl.BlockSpec((3, c2, 3 * c2), lambda i: (0, 0, 0)),
                pl.BlockSpec((3, c2, 3 * c2), lambda i: (0, 0, 0)),
                pl.BlockSpec((c2, 1), lambda i: (0, 0)),
                pl.BlockSpec((c2, 1), lambda i: (0, 0)),
                pl.BlockSpec((1, m + 2 * wo), lambda i: (0, 0)),
                pl.BlockSpec((1, m + 2 * wo), lambda i: (0, 0)),
                pl.BlockSpec((1, cin, hin * win), lambda i: (i, 0, 0)),
            ],
            out_specs=pl.BlockSpec((1, c2, m), lambda i: (i, 0, 0)),
            scratch_shapes=[
                pltpu.VMEM((cin, hin * wo), jnp.bfloat16),
                pltpu.VMEM((3 * cin, mp), jnp.bfloat16),
                pltpu.VMEM((3 * c2, mp), jnp.bfloat16),
                pltpu.VMEM((3 * c2, mp), jnp.bfloat16),
            ],
        ),
        compiler_params=pltpu.CompilerParams(
            dimension_semantics=("parallel",)),
        cost_estimate=pl.CostEstimate(
            flops=flops, transcendentals=0, bytes_accessed=bytes_accessed),
    )(awt, wu9, w19, w29, sh1, sh2, mask_l, mask_r, x2)
    return out.reshape(n, c2, ho, wo)


# speedup vs baseline: 1.1322x; 1.0050x over previous
import functools

import numpy as np
import jax
import jax.numpy as jnp
from jax.experimental import pallas as pl
from jax.experimental.pallas import tpu as pltpu


def _interp_consts(n_in, n_out):
    scale = np.float32(np.float32(n_in - 1) / np.float32(n_out - 1))
    pos = np.arange(n_out, dtype=np.float32) * scale
    lo = np.clip(np.floor(pos), 0, n_in - 1).astype(np.int32)
    hi = np.minimum(lo + 1, n_in - 1)
    frac = (pos - lo.astype(np.float32)).astype(np.float32)
    return lo, hi, frac


def _interp_matrix_np(n_in, n_out):
    lo, hi, frac = _interp_consts(n_in, n_out)
    a = np.zeros((n_out, n_in), np.float32)
    for i in range(n_out):
        a[i, lo[i]] += 1.0 - frac[i]
        a[i, hi[i]] += frac[i]
    return a


def _fused_kernel(awt_ref, wu_ref, w1_ref, w2_ref, sh1_ref, sh2_ref,
                  ml_ref, mr_ref, x_ref, o_ref,
                  xw_ref, up_ref, ypad_ref, hpad_ref,
                  *, hin, win, wo, ho, m, padl, g, lo, hi, frac):
    cin = wu_ref.shape[2] // 3

    for ref in (up_ref, ypad_ref, hpad_ref):
        c = ref.shape[0] // 3
        ref[c:2 * c, :padl] = jnp.zeros_like(ref[c:2 * c, :padl])
        ref[c:2 * c, padl + m:] = jnp.zeros_like(ref[c:2 * c, padl + m:])

    xsrc = x_ref[0].astype(jnp.bfloat16)
    for k in range(0, hin, g):
        blk = jnp.dot(xsrc[:, k * win:(k + g) * win], awt_ref[...],
                      preferred_element_type=jnp.float32)
        xw_ref[:, k * wo:(k + g) * wo] = blk.astype(jnp.bfloat16)
    for i in range(ho):
        l, h, f = lo[i], hi[i], frac[i]
        r0 = xw_ref[:, l * wo:(l + 1) * wo]
        if f == 0.0 or h == l:
            row = r0
        else:
            r1 = xw_ref[:, h * wo:(h + 1) * wo]
            row = r0 * jnp.bfloat16(1.0 - f) + r1 * jnp.bfloat16(f)
        up_ref[cin:2 * cin, padl + i * wo: padl + (i + 1) * wo] = row

    mask_l = ml_ref[...]
    mask_r = mr_ref[...]

    def conv3x3(p3_ref, w3_ref):
        c = w3_ref.shape[2] // 3
        a, b = padl - wo, padl + m + wo
        p3_ref[:c, a:b] = p3_ref[c:2 * c, a - 1:b - 1] * mask_l
        p3_ref[2 * c:3 * c, a:b] = p3_ref[c:2 * c, a + 1:b + 1] * mask_r
        acc = None
        for dy in range(3):
            s = padl + (dy - 1) * wo
            part = jnp.dot(w3_ref[dy], p3_ref[:, s:s + m],
                           preferred_element_type=jnp.float32)
            acc = part if acc is None else acc + part
        return acc

    c2 = w1_ref.shape[1]
    y = jnp.maximum(conv3x3(up_ref, wu_ref), 0.0)
    ypad_ref[c2:2 * c2, padl:padl + m] = y.astype(jnp.bfloat16)
    h1 = jnp.maximum(conv3x3(ypad_ref, w1_ref) + sh1_ref[...], 0.0)
    hpad_ref[c2:2 * c2, padl:padl + m] = h1.astype(jnp.bfloat16)
    out = jnp.maximum(conv3x3(hpad_ref, w2_ref) + sh2_ref[...] + y, 0.0)
    o_ref[0] = out.astype(o_ref.dtype)


def kernel(x, w_up, w1, b1, bn1_gamma, bn1_beta, bn1_mean, bn1_var,
           w2, b2, bn2_gamma, bn2_beta, bn2_mean, bn2_var):
    eps = 1e-5
    n, cin, hin, win = x.shape
    c2 = w_up.shape[0]
    ho, wo = 2 * hin, 2 * win
    m = ho * wo
    padl = ((wo + 1 + 127) // 128) * 128
    mp = m + 2 * padl

    s1 = bn1_gamma / jnp.sqrt(bn1_var + eps)
    sh1 = ((b1 - bn1_mean) * s1 + bn1_beta).reshape(c2, 1).astype(jnp.float32)
    s2 = bn2_gamma / jnp.sqrt(bn2_var + eps)
    sh2 = ((b2 - bn2_mean) * s2 + bn2_beta).reshape(c2, 1).astype(jnp.float32)

    def tap_major(w, scale=None):
        if scale is not None:
            w = w * scale[:, None, None, None]
        o, i = w.shape[0], w.shape[1]
        return jnp.transpose(w, (2, 0, 3, 1)).reshape(3, o, 3 * i) \
            .astype(jnp.bfloat16)

    wu9 = tap_major(w_up)
    w19 = tap_major(w1, s1)
    w29 = tap_major(w2, s2)

    g = max(1, 256 // win)
    while hin % g:
        g //= 2
    aw1 = _interp_matrix_np(win, wo).T
    awb = np.zeros((g * win, g * wo), np.float32)
    for j in range(g):
        awb[j * win:(j + 1) * win, j * wo:(j + 1) * wo] = aw1
    awt = jnp.asarray(awb, jnp.bfloat16)
    lo, hi, frac = _interp_consts(hin, ho)

    col = jnp.arange(m + 2 * wo, dtype=jnp.int32) % wo
    mask_l = (col >= 1).astype(jnp.bfloat16).reshape(1, m + 2 * wo)
    mask_r = (col < wo - 1).astype(jnp.bfloat16).reshape(1, m + 2 * wo)

    x2 = x.reshape(n, cin, hin * win)

    flops = 2 * n * m * 9 * (c2 * cin + 2 * c2 * c2) + 4 * n * cin * m
    bytes_accessed = 4 * (n * cin * hin * win + n * c2 * m) \
        + 2 * 9 * (c2 * cin + 2 * c2 * c2) + 2 * 2 * m

    out = pl.pallas_call(
        functools.partial(_fused_kernel, hin=hin, win=win, wo=wo, ho=ho, m=m,
                          padl=padl, g=g,
                          lo=tuple(int(v) for v in lo),
                          hi=tuple(int(v) for v in hi),
                          frac=tuple(float(v) for v in frac)),
        out_shape=jax.ShapeDtypeStruct((n, c2, m), x.dtype),
        grid_spec=pltpu.PrefetchScalarGridSpec(
            num_scalar_prefetch=0,
            grid=(n,),
            in_specs=[
                pl.BlockSpec((g * win, g * wo), lambda i: (0, 0)),
                pl.BlockSpec((3, c2, 3 * cin), lambda i: (0, 0, 0)),
                pl.BlockSpec((3, c2, 3 * c2), lambda i: (0, 0, 0)),
                pl.BlockSpec((3, c2, 3 * c2), lambda i: (0, 0, 0)),
                pl.BlockSpec((c2, 1), lambda i: (0, 0)),
                pl.BlockSpec((c2, 1), lambda i: (0, 0)),
                pl.BlockSpec((1, m + 2 * wo), lambda i: (0, 0)),
                pl.BlockSpec((1, m + 2 * wo), lambda i: (0, 0)),
                pl.BlockSpec((1, cin, hin * win), lambda i: (i, 0, 0)),
            ],
            out_specs=pl.BlockSpec((1, c2, m), lambda i: (i, 0, 0)),
            scratch_shapes=[
                pltpu.VMEM((cin, hin * wo), jnp.bfloat16),
                pltpu.VMEM((3 * cin, mp), jnp.bfloat16),
                pltpu.VMEM((3 * c2, mp), jnp.bfloat16),
                pltpu.VMEM((3 * c2, mp), jnp.bfloat16),
            ],
        ),
        compiler_params=pltpu.CompilerParams(
            dimension_semantics=("parallel",)),
        cost_estimate=pl.CostEstimate(
            flops=flops, transcendentals=0, bytes_accessed=bytes_accessed),
    )(awt, wu9, w19, w29, sh1, sh2, mask_l, mask_r, x2)
    return out.reshape(n, c2, ho, wo)
